# baseline (speedup 1.0000x reference)
import jax
import jax.numpy as jnp
from jax.experimental import pallas as pl
from jax.experimental.pallas import tpu as pltpu

AUG_THRESHOLD = 1024
BR = 4
BIG = 2**30


def _aug_body(lens_ref, ptab_ref, x_ref, o_ref):
    g = pl.program_id(0)
    ptab = ptab_ref[...]
    for j in range(BR):
        ln = lens_ref[g * BR + j]
        lim = jnp.where(ln > AUG_THRESHOLD, ln, 0)
        o_ref[j, :, :] = jnp.where(ptab[0] < lim, 0.0, x_ref[j, :, :])


def kernel(sequences, seq_lens):
    B, L, D = sequences.shape
    pos = jnp.arange(L, dtype=jnp.int32)
    ptab = jnp.where(pos % 10 == 0, pos, BIG)[None, :, None]
    grid = (B // BR,)
    out = pl.pallas_call(
        _aug_body,
        grid_spec=pltpu.PrefetchScalarGridSpec(
            num_scalar_prefetch=1,
            grid=grid,
            in_specs=[
                pl.BlockSpec((1, L, 1), lambda g, lens: (0, 0, 0)),
                pl.BlockSpec((BR, L, D), lambda g, lens: (g, 0, 0)),
            ],
            out_specs=pl.BlockSpec((BR, L, D), lambda g, lens: (g, 0, 0)),
        ),
        out_shape=jax.ShapeDtypeStruct((B, L, D), sequences.dtype),
        compiler_params=pltpu.CompilerParams(
            dimension_semantics=("parallel",),
            vmem_limit_bytes=110 * 1024 * 1024,
        ),
    )(seq_lens, ptab, sequences)
    return out, seq_lens


# speedup vs baseline: 1.0602x; 1.0602x over previous
import jax
import jax.numpy as jnp
from jax.experimental import pallas as pl
from jax.experimental.pallas import tpu as pltpu

AUG_THRESHOLD = 1024
BIG = 2**30
NBUF_IN = 4


def _make_outer(B, L, D):
    def outer(lens_ref, ptab_ref, x_hbm, o_hbm):
        def inner(x_ref, o_ref):
            b = pl.program_id(0)
            ln = lens_ref[b]
            lim = jnp.where(ln > AUG_THRESHOLD, ln, 0)
            mask = ptab_ref[...] < lim
            o_ref[...] = jnp.where(mask, 0.0, x_ref[...])

        pipeline = pltpu.emit_pipeline(
            inner,
            grid=(B,),
            in_specs=[
                pl.BlockSpec(
                    (1, L, D),
                    lambda b: (b, 0, 0),
                    pipeline_mode=pl.Buffered(
                        buffer_count=NBUF_IN, use_lookahead=True
                    ),
                )
            ],
            out_specs=[pl.BlockSpec((1, L, D), lambda b: (b, 0, 0))],
        )
        pipeline(x_hbm, o_hbm)

    return outer


def kernel(sequences, seq_lens):
    B, L, D = sequences.shape
    pos = jnp.arange(L, dtype=jnp.int32)
    ptab = jnp.where(pos % 10 == 0, pos, BIG)[None, :, None]
    out = pl.pallas_call(
        _make_outer(B, L, D),
        grid_spec=pltpu.PrefetchScalarGridSpec(
            num_scalar_prefetch=1,
            grid=(1,),
            in_specs=[
                pl.BlockSpec(memory_space=pltpu.VMEM),
                pl.BlockSpec(memory_space=pltpu.HBM),
            ],
            out_specs=pl.BlockSpec(memory_space=pltpu.HBM),
        ),
        out_shape=jax.ShapeDtypeStruct((B, L, D), sequences.dtype),
    )(seq_lens, ptab, sequences)
    return out, seq_lens
